# baseline (speedup 1.0000x reference)
import jax
import jax.numpy as jnp
from jax import lax
from jax.experimental import pallas as pl
from jax.experimental.pallas import tpu as pltpu
from jax.experimental.pallas import tpu_sc as plsc

B, K, H, W = 1024, 3, 8, 8
HW = H * W
D = 128
NC, NS, L = 2, 16, 16
NW = NC * NS
BPW = B // NW
KHW = K * HW
CD = D // L
IRW = BPW * KHW // 128


def _sc_body(inp_hbm, table_hbm, out_hbm,
             idx_v, acc0, acc1, accT0, accT1,
             gsem0, gsem1, osem0, osem1):
    wid = lax.axis_index("s") * NC + lax.axis_index("c")
    b0 = wid * BPW
    pltpu.sync_copy(inp_hbm.at[pl.ds(wid * IRW, IRW)], idx_v)

    lanes = lax.iota(jnp.int32, L)
    zeros16 = jnp.zeros((L,), jnp.float32)
    row2_vecs = [c * (L // 2) + (lanes // 2) for c in range(CD)]
    col2_base = (lanes % 2) * HW

    def zero_acc(acc):
        @plsc.parallel_loop(0, HW, 1, unroll=2)
        def _(r):
            for c in range(CD):
                acc[r, pl.ds(c * L, L)] = zeros16

    def fire_gathers(j, acc, gsem):
        for k in range(K):
            base = j * KHW + k * HW
            pltpu.async_copy(
                table_hbm.at[idx_v.at[base // 128, pl.ds(base % 128, HW)]],
                acc, gsem, add=True)

    def wait_gathers(acc, gsem):
        for k in range(K):
            pltpu.make_async_copy(
                table_hbm.at[idx_v.at[0, pl.ds(0, HW)]], acc, gsem).wait()

    def transpose(acc, accT):
        @plsc.parallel_loop(0, HW, 1, unroll=2)
        def _(hw):
            colv = col2_base + hw
            for c in range(CD):
                plsc.store_scatter(accT, [row2_vecs[c], colv],
                                   acc[hw, pl.ds(c * L, L)])

    bufs = ((acc0, accT0, gsem0, osem0), (acc1, accT1, gsem1, osem1))

    zero_acc(acc0)
    zero_acc(acc1)
    fire_gathers(0, acc0, gsem0)
    fire_gathers(1, acc1, gsem1)

    def pipe(t, c2):
        for p, (acc, accT, gsem, osem) in enumerate(bufs):
            j = t * 2 + p
            wait_gathers(acc, gsem)

            @pl.when(j >= 2)
            def _():
                pltpu.make_async_copy(accT, out_hbm.at[pl.ds(0, HW)],
                                      osem).wait()

            transpose(acc, accT)
            zero_acc(acc)

            @pl.when(j + 2 < BPW)
            def _():
                fire_gathers(j + 2, acc, gsem)

            pltpu.async_copy(accT, out_hbm.at[pl.ds((b0 + j) * HW, HW)],
                             osem)
        return c2

    lax.fori_loop(0, BPW // 2, pipe, 0)
    pltpu.make_async_copy(accT0, out_hbm.at[pl.ds(0, HW)], osem0).wait()
    pltpu.make_async_copy(accT1, out_hbm.at[pl.ds(0, HW)], osem1).wait()


def kernel(inputs, table):
    inp2 = inputs.reshape(B * K * HW // 128, 128)
    mesh = plsc.VectorSubcoreMesh(
        core_axis_name="c", subcore_axis_name="s",
        num_cores=NC, num_subcores=NS,
    )
    out = pl.kernel(
        _sc_body,
        out_type=jax.ShapeDtypeStruct((B * D * HW // 128, 128), jnp.float32),
        mesh=mesh,
        scratch_types=[
            pltpu.VMEM((IRW, 128), jnp.int32),
            pltpu.VMEM((HW, D), jnp.float32),
            pltpu.VMEM((HW, D), jnp.float32),
            pltpu.VMEM((HW, D), jnp.float32),
            pltpu.VMEM((HW, D), jnp.float32),
            pltpu.SemaphoreType.DMA,
            pltpu.SemaphoreType.DMA,
            pltpu.SemaphoreType.DMA,
            pltpu.SemaphoreType.DMA,
        ],
        compiler_params=pltpu.CompilerParams(needs_layout_passes=False),
    )(inp2, table)
    return out.reshape(B, D, H, W)


# speedup vs baseline: 7.7399x; 7.7399x over previous
import jax
import jax.numpy as jnp
from jax import lax
from jax.experimental import pallas as pl
from jax.experimental.pallas import tpu as pltpu
from jax.experimental.pallas import tpu_sc as plsc

B, K, H, W = 1024, 3, 8, 8
HW = H * W
D = 128
NC, NS, L = 2, 16, 16
NW = NC * NS
BPW = B // NW
KHW = K * HW
NBUF = 4


def _sc_body(inp_hbm, table_hbm, out_hbm, idx_v, idxT_v, *accs_and_sems):
    accs = accs_and_sems[:NBUF]
    gsems = accs_and_sems[NBUF:2 * NBUF]
    osems = accs_and_sems[2 * NBUF:]
    wid = lax.axis_index("s") * NC + lax.axis_index("c")
    b0 = wid * BPW

    pltpu.sync_copy(inp_hbm.at[:, pl.ds((wid // 4) * 128, 128)], idx_v)

    lanes = lax.iota(jnp.int32, L)
    sub = (wid % 4) * BPW

    @plsc.parallel_loop(0, KHW, 1, unroll=2)
    def _(r):
        rv = jnp.full((L,), r, dtype=jnp.int32)
        plsc.store_scatter(idxT_v, [lanes, rv], idx_v[r, pl.ds(sub, L)])
        plsc.store_scatter(idxT_v, [L + lanes, rv],
                           idx_v[r, pl.ds(sub + L, L)])

    def fire_g0(j, p):
        pltpu.async_copy(table_hbm.at[idxT_v.at[j, pl.ds(0, HW)]],
                         accs[p], gsems[p])

    def fire_adds(j, p):
        for k in range(1, K):
            pltpu.async_copy(table_hbm.at[idxT_v.at[j, pl.ds(k * HW, HW)]],
                             accs[p], gsems[p], add=True)

    def wait_g(p, n):
        for _ in range(n):
            pltpu.make_async_copy(table_hbm.at[idxT_v.at[0, pl.ds(0, HW)]],
                                  accs[p], gsems[p]).wait()

    def fire_out(j, p):
        pltpu.async_copy(accs[p],
                         out_hbm.at[pl.ds((b0 + j) * HW, HW)], osems[p])

    def wait_out(p):
        pltpu.make_async_copy(accs[p], out_hbm.at[pl.ds(0, HW)],
                              osems[p]).wait()

    fire_g0(0, 0)
    fire_g0(1, 1)
    wait_g(0, 1)
    fire_adds(0, 0)

    def pipe(t, c2):
        for p in range(NBUF):
            j = t * NBUF + p
            jj = j + 2
            pj = (p + 2) % NBUF

            @pl.when(jj < BPW)
            def _():
                @pl.when(jj >= NBUF)
                def _():
                    wait_out(pj)
                fire_g0(jj, pj)

            ja = j + 1
            pa = (p + 1) % NBUF

            @pl.when(ja < BPW)
            def _():
                wait_g(pa, 1)
                fire_adds(ja, pa)

            wait_g(p, 2)
            fire_out(j, p)
        return c2

    lax.fori_loop(0, BPW // NBUF, pipe, 0)
    for j in range(BPW - NBUF, BPW):
        wait_out(j % NBUF)


def kernel(inputs, table):
    inp2 = inputs.transpose(1, 2, 3, 0).reshape(KHW, B)
    mesh = plsc.VectorSubcoreMesh(
        core_axis_name="c", subcore_axis_name="s",
        num_cores=NC, num_subcores=NS,
    )
    scratch = (
        [pltpu.VMEM((KHW, 128), jnp.int32),
         pltpu.VMEM((BPW, 256), jnp.int32)]
        + [pltpu.VMEM((HW, D), jnp.float32) for _ in range(NBUF)]
        + [pltpu.SemaphoreType.DMA for _ in range(2 * NBUF)]
    )
    out = pl.kernel(
        _sc_body,
        out_type=jax.ShapeDtypeStruct((B * HW, D), jnp.float32),
        mesh=mesh,
        scratch_types=scratch,
        compiler_params=pltpu.CompilerParams(needs_layout_passes=False),
    )(inp2, table)
    return out.reshape(B, H, W, D).transpose(0, 3, 1, 2)


# speedup vs baseline: 7.7859x; 1.0060x over previous
import jax
import jax.numpy as jnp
from jax import lax
from jax.experimental import pallas as pl
from jax.experimental.pallas import tpu as pltpu
from jax.experimental.pallas import tpu_sc as plsc

B, K, H, W = 1024, 3, 8, 8
HW = H * W
D = 128
NC, NS, L = 2, 16, 16
NW = NC * NS
BPW = B // NW
KHW = K * HW
NBUF = 8
DG0 = 4
DADD = 2


def _sc_body(inp_hbm, table_hbm, out_hbm, idx_v, idxT_v, *accs_and_sems):
    accs = accs_and_sems[:NBUF]
    gsems = accs_and_sems[NBUF:2 * NBUF]
    osems = accs_and_sems[2 * NBUF:]
    wid = lax.axis_index("s") * NC + lax.axis_index("c")
    b0 = wid * BPW

    pltpu.sync_copy(inp_hbm.at[:, pl.ds((wid // 4) * 128, 128)], idx_v)

    lanes = lax.iota(jnp.int32, L)
    sub = (wid % 4) * BPW

    @plsc.parallel_loop(0, KHW, 1, unroll=2)
    def _(r):
        rv = jnp.full((L,), r, dtype=jnp.int32)
        plsc.store_scatter(idxT_v, [lanes, rv], idx_v[r, pl.ds(sub, L)])
        plsc.store_scatter(idxT_v, [L + lanes, rv],
                           idx_v[r, pl.ds(sub + L, L)])

    def fire_g0(j, p):
        pltpu.async_copy(table_hbm.at[idxT_v.at[j, pl.ds(0, HW)]],
                         accs[p], gsems[p])

    def fire_adds(j, p):
        for k in range(1, K):
            pltpu.async_copy(table_hbm.at[idxT_v.at[j, pl.ds(k * HW, HW)]],
                             accs[p], gsems[p], add=True)

    def wait_g(p, n):
        for _ in range(n):
            pltpu.make_async_copy(table_hbm.at[idxT_v.at[0, pl.ds(0, HW)]],
                                  accs[p], gsems[p]).wait()

    def fire_out(j, p):
        pltpu.async_copy(accs[p],
                         out_hbm.at[pl.ds((b0 + j) * HW, HW)], osems[p])

    def wait_out(p):
        pltpu.make_async_copy(accs[p], out_hbm.at[pl.ds(0, HW)],
                              osems[p]).wait()

    for j in range(DG0):
        fire_g0(j, j)
    for j in range(DADD):
        wait_g(j, 1)
        fire_adds(j, j)

    def pipe(t, c2):
        for p in range(NBUF):
            j = t * NBUF + p
            jj = j + DG0
            pj = (p + DG0) % NBUF

            @pl.when(jj < BPW)
            def _():
                @pl.when(jj >= NBUF)
                def _():
                    wait_out(pj)
                fire_g0(jj, pj)

            ja = j + DADD
            pa = (p + DADD) % NBUF

            @pl.when(ja < BPW)
            def _():
                wait_g(pa, 1)
                fire_adds(ja, pa)

            wait_g(p, 2)
            fire_out(j, p)
        return c2

    lax.fori_loop(0, BPW // NBUF, pipe, 0)
    for j in range(BPW - NBUF, BPW):
        wait_out(j % NBUF)


def kernel(inputs, table):
    inp2 = inputs.transpose(1, 2, 3, 0).reshape(KHW, B)
    mesh = plsc.VectorSubcoreMesh(
        core_axis_name="c", subcore_axis_name="s",
        num_cores=NC, num_subcores=NS,
    )
    scratch = (
        [pltpu.VMEM((KHW, 128), jnp.int32),
         pltpu.VMEM((BPW, 256), jnp.int32)]
        + [pltpu.VMEM((HW, D), jnp.float32) for _ in range(NBUF)]
        + [pltpu.SemaphoreType.DMA for _ in range(2 * NBUF)]
    )
    out = pl.kernel(
        _sc_body,
        out_type=jax.ShapeDtypeStruct((B * HW, D), jnp.float32),
        mesh=mesh,
        scratch_types=scratch,
        compiler_params=pltpu.CompilerParams(needs_layout_passes=False),
    )(inp2, table)
    return out.reshape(B, H, W, D).transpose(0, 3, 1, 2)
